# baseline (speedup 1.0000x reference)
import functools

import jax
import jax.numpy as jnp
from jax import lax
from jax.experimental import pallas as pl
from jax.experimental.pallas import tpu as pltpu
from jax.experimental.pallas import tpu_sc as plsc

N = 10000
E = 320000
F_IN = 128
H = 64
HP = 128
C = 10

NC = 2
NS = 16
NW = NC * NS
CH = 128
NCH = 80
NB = 8
NBLK = NCH // NB
EPT = CH * NCH
EPAD = EPT * NW
JUNK = 112
NACC = N + JUNK
ZR = NACC // NS


NBUF = 4
LAG = NBUF // 2
YI = 632
YI15 = 520
YS = N // NS


def _seg_body(y_h, src_h, dst_h, out_h, srcb, dstb, rows_v, acc, ysh, *sems_all):
    semg = sems_all[:NBUF]
    sems = sems_all[NBUF:2 * NBUF]
    semi = sems_all[2 * NBUF]
    c = lax.axis_index("c")
    s = lax.axis_index("s")
    wid = c * NS + s
    pltpu.sync_copy(src_h.at[wid, pl.ds(0, NB)], srcb.at[0])
    pltpu.sync_copy(dst_h.at[wid, pl.ds(0, NB)], dstb.at[0])
    pltpu.sync_copy(y_h.at[pl.ds(s * YS, YS)], ysh.at[pl.ds(s * YS, YS)])

    @pl.when(s < NS - 1)
    def _():
        pltpu.sync_copy(y_h.at[pl.ds(s * YI, YI)], acc.at[pl.ds(s * YI, YI)])

    @pl.when(s == NS - 1)
    def _():
        pltpu.sync_copy(y_h.at[pl.ds((NS - 1) * YI, YI15)],
                        acc.at[pl.ds((NS - 1) * YI, YI15)])

    plsc.subcore_barrier()

    def gsrc(k):
        return y_h if (k % NB) in (0, 2, 3, 5, 7) else ysh

    def scat(bb, k, b):
        return pltpu.make_async_copy(rows_v.at[b], acc.at[dstb.at[bb, k]],
                                     sems[b])

    for k in range(LAG):
        pltpu.async_copy(gsrc(k).at[srcb.at[0, k]], rows_v.at[k], semg[k])

    def block(blk, carry):
        bb = blk % 3
        nbb = (blk + 1) % 3
        for k in range(NB):
            j = blk * NB + k
            b = k % NBUF
            b2 = (k - LAG) % NBUF
            if k == 0:
                @pl.when(blk + 1 < NBLK)
                def _():
                    nxt = (blk + 1) * NB
                    pltpu.make_async_copy(
                        src_h.at[wid, pl.ds(nxt, NB)], srcb.at[nbb], semi).start()
                    pltpu.make_async_copy(
                        dst_h.at[wid, pl.ds(nxt, NB)], dstb.at[nbb], semi).start()
            pltpu.make_async_copy(
                gsrc(k).at[srcb.at[bb, k]], rows_v.at[b], semg[b]).wait()
            pltpu.async_copy(rows_v.at[b], acc.at[dstb.at[bb, k]],
                             sems[b], add=True)
            if k < LAG:
                @pl.when(j >= LAG)
                def _():
                    scat((blk - 1) % 3, NB - LAG + k, b2).wait()
            else:
                scat(bb, k - LAG, b2).wait()
            if k == NB - LAG - 1:
                @pl.when(blk + 1 < NBLK)
                def _():
                    nxt = (blk + 1) * NB
                    pltpu.make_async_copy(
                        src_h.at[wid, pl.ds(nxt, NB)], srcb.at[nbb], semi).wait()
                    pltpu.make_async_copy(
                        dst_h.at[wid, pl.ds(nxt, NB)], dstb.at[nbb], semi).wait()
            if k + LAG < NB:
                pltpu.async_copy(gsrc(k + LAG).at[srcb.at[bb, k + LAG]],
                                 rows_v.at[b2], semg[b2])
            else:
                @pl.when(j + LAG < NCH)
                def _():
                    pltpu.async_copy(gsrc(k + LAG).at[srcb.at[nbb, k + LAG - NB]],
                                     rows_v.at[b2], semg[b2])
        return carry

    lax.fori_loop(0, NBLK, block, 0)
    bl = (NBLK - 1) % 3
    for k in range(NB - LAG, NB):
        scat(bl, k, k % NBUF).wait()
    plsc.subcore_barrier()
    pltpu.sync_copy(acc.at[pl.ds(s * ZR, ZR)], out_h.at[c, pl.ds(s * ZR, ZR)])


_seg_kernel = functools.partial(
    pl.kernel,
    out_type=jax.ShapeDtypeStruct((NC, NACC, H), jnp.float32),
    mesh=plsc.VectorSubcoreMesh(core_axis_name="c", subcore_axis_name="s"),
    compiler_params=pltpu.CompilerParams(use_tc_tiling_on_sc=False),
    scratch_types=[
        pltpu.VMEM((3, NB, CH), jnp.int32),
        pltpu.VMEM((3, NB, CH), jnp.int32),
        pltpu.VMEM((NBUF, CH, H), jnp.float32),
        pltpu.VMEM_SHARED((NACC, H), jnp.float32),
        pltpu.VMEM_SHARED((N, H), jnp.float32),
    ] + [pltpu.SemaphoreType.DMA] * (2 * NBUF + 1),
)(_seg_body)


N2 = N // 2
NACC2 = NACC // 2


def _bdiag(w_ref):
    w = w_ref[...]
    z = jnp.zeros_like(w)
    return jnp.concatenate(
        [jnp.concatenate([w, z], axis=1), jnp.concatenate([z, w], axis=1)],
        axis=0)


def _dup(v_ref):
    v = v_ref[...]
    return jnp.concatenate([v, v], axis=1)


def _pair_bn(u, g_ref, be_ref):
    mu2 = jnp.mean(u, axis=0, keepdims=True)
    e2 = jnp.mean(u * u, axis=0, keepdims=True)
    mu = 0.5 * (mu2[:, :H] + mu2[:, H:])
    var = 0.5 * (e2[:, :H] + e2[:, H:]) - mu * mu
    muf = jnp.concatenate([mu, mu], axis=1)
    varf = jnp.concatenate([var, var], axis=1)
    return (u - muf) * lax.rsqrt(varf + 1e-5) * _dup(g_ref) + _dup(be_ref)


def _mm_body(x_ref, w_ref, o_ref):
    o_ref[...] = jnp.dot(x_ref[...], w_ref[...],
                         preferred_element_type=jnp.float32)


def _mid_body(y_ref, p_ref, b1_ref, w2_ref, b2_ref, g_ref, be_ref,
              w1n_ref, o_ref):
    h = p_ref[0, :N2, :] + p_ref[1, :N2, :] - y_ref[...] + _dup(b1_ref)
    h = jnp.maximum(h, 0.0)
    u = jnp.dot(h, _bdiag(w2_ref),
                preferred_element_type=jnp.float32) + _dup(b2_ref)
    u = jnp.maximum(u, 0.0)
    hb = _pair_bn(u, g_ref, be_ref)
    o_ref[...] = jnp.dot(hb, _bdiag(w1n_ref), preferred_element_type=jnp.float32)


def _head_body(y_ref, p_ref, b1_ref, w2_ref, b2_ref, g_ref, be_ref,
               l1w_ref, l1b_ref, l2w_ref, l2b_ref, o_ref):
    h = p_ref[0, :N2, :] + p_ref[1, :N2, :] - y_ref[...] + _dup(b1_ref)
    h = jnp.maximum(h, 0.0)
    u = jnp.dot(h, _bdiag(w2_ref),
                preferred_element_type=jnp.float32) + _dup(b2_ref)
    u = jnp.maximum(u, 0.0)
    hb = _pair_bn(u, g_ref, be_ref)
    t = jnp.maximum(
        jnp.dot(hb, _bdiag(l1w_ref), preferred_element_type=jnp.float32)
        + _dup(l1b_ref), 0.0)
    l2w = l2w_ref[...]
    z = jnp.zeros_like(l2w)
    l2d = jnp.concatenate(
        [jnp.concatenate([l2w, z], axis=1), jnp.concatenate([z, l2w], axis=1)],
        axis=0)
    logits = jnp.dot(t, l2d, preferred_element_type=jnp.float32) + _dup(l2b_ref)
    le, lo = logits[:, :C], logits[:, C:]
    me = jnp.max(le, axis=-1, keepdims=True)
    mo = jnp.max(lo, axis=-1, keepdims=True)
    lse_e = me + jnp.log(jnp.sum(jnp.exp(le - me), axis=-1, keepdims=True))
    lse_o = mo + jnp.log(jnp.sum(jnp.exp(lo - mo), axis=-1, keepdims=True))
    o_ref[...] = logits - jnp.concatenate(
        [jnp.broadcast_to(lse_e, (N2, C)), jnp.broadcast_to(lse_o, (N2, C))],
        axis=1)


def _tc(body, out_shape, *args):
    return pl.pallas_call(body, out_shape=out_shape)(*args)


def _padw(w):
    return jnp.pad(w, ((0, 0), (0, HP - H)))


def kernel(x, edge_index, c0_W1, c0_b1, c0_W2, c0_b2, c0_g, c0_be,
           c1_W1, c1_b1, c1_W2, c1_b2, c1_g, c1_be,
           c2_W1, c2_b1, c2_W2, c2_b2, c2_g, c2_be,
           l1_W, l1_b, l2_W, l2_b):
    src, dst = edge_index[0], edge_index[1]
    pad = EPAD - E
    ar = jnp.arange(pad, dtype=jnp.int32)
    src_r = jnp.concatenate([src, ar % N]).reshape(NW, NCH, CH)
    dst_r = jnp.concatenate([dst, N + (ar % JUNK)]).reshape(NW, NCH, CH)

    f32 = jnp.float32
    b1s = [c0_b1.reshape(1, H), c1_b1.reshape(1, H), c2_b1.reshape(1, H)]
    b2s = [c0_b2.reshape(1, H), c1_b2.reshape(1, H), c2_b2.reshape(1, H)]
    gs = [c0_g.reshape(1, H), c1_g.reshape(1, H), c2_g.reshape(1, H)]
    bes = [c0_be.reshape(1, H), c1_be.reshape(1, H), c2_be.reshape(1, H)]
    W2s = [c0_W2, c1_W2, c2_W2]

    y0 = _tc(_mm_body, jax.ShapeDtypeStruct((N, H), f32), x, c0_W1)
    y2 = y0.reshape(N2, 2 * H)

    for i in range(3):
        p = _seg_kernel(y2.reshape(N, H), src_r, dst_r)
        p2 = p.reshape(NC, NACC2, 2 * H)
        if i < 2:
            w1n = c1_W1 if i == 0 else c2_W1
            y2 = _tc(_mid_body, jax.ShapeDtypeStruct((N2, 2 * H), f32),
                     y2, p2, b1s[i], W2s[i], b2s[i], gs[i], bes[i], w1n)
        else:
            out2 = _tc(_head_body, jax.ShapeDtypeStruct((N2, 2 * C), f32),
                       y2, p2, b1s[i], W2s[i], b2s[i], gs[i], bes[i],
                       l1_W, l1_b.reshape(1, H), l2_W, l2_b.reshape(1, C))
    return out2.reshape(N, C)


# speedup vs baseline: 1.2524x; 1.2524x over previous
import functools

import jax
import jax.numpy as jnp
from jax import lax
from jax.experimental import pallas as pl
from jax.experimental.pallas import tpu as pltpu
from jax.experimental.pallas import tpu_sc as plsc

N = 10000
E = 320000
F_IN = 128
H = 64
HP = 128
C = 10

NC = 2
NS = 16
NW = NC * NS
CH = 128
NCH = 80
NB = 8
NBLK = NCH // NB
EPT = CH * NCH
EPAD = EPT * NW
JUNK = 112
NACC = N + JUNK
ZR = NACC // NS


NBUF = 8
LAG = NBUF // 2
YI = 632
YI15 = 520


def _seg_body(y_h, src_h, dst_h, out_h, srcb, dstb, rows_v, acc, *sems_all):
    semg = sems_all[:NBUF]
    sems = sems_all[NBUF:2 * NBUF]
    semi = sems_all[2 * NBUF]
    c = lax.axis_index("c")
    s = lax.axis_index("s")
    wid = c * NS + s
    pltpu.sync_copy(src_h.at[wid, pl.ds(0, NB)], srcb.at[0])
    pltpu.sync_copy(dst_h.at[wid, pl.ds(0, NB)], dstb.at[0])

    @pl.when(s < NS - 1)
    def _():
        pltpu.sync_copy(y_h.at[pl.ds(s * YI, YI)], acc.at[pl.ds(s * YI, YI)])

    @pl.when(s == NS - 1)
    def _():
        pltpu.sync_copy(y_h.at[pl.ds((NS - 1) * YI, YI15)],
                        acc.at[pl.ds((NS - 1) * YI, YI15)])

    plsc.subcore_barrier()

    def scat(bb, k, b):
        return pltpu.make_async_copy(rows_v.at[b], acc.at[dstb.at[bb, k]],
                                     sems[b])

    for k in range(LAG):
        pltpu.async_copy(y_h.at[srcb.at[0, k]], rows_v.at[k], semg[k])

    def block(blk, carry):
        bb = blk % 3
        nbb = (blk + 1) % 3
        for k in range(NB):
            j = blk * NB + k
            b = k % NBUF
            b2 = (k - LAG) % NBUF
            if k == 0:
                @pl.when(blk + 1 < NBLK)
                def _():
                    nxt = (blk + 1) * NB
                    pltpu.make_async_copy(
                        src_h.at[wid, pl.ds(nxt, NB)], srcb.at[nbb], semi).start()
                    pltpu.make_async_copy(
                        dst_h.at[wid, pl.ds(nxt, NB)], dstb.at[nbb], semi).start()
            pltpu.make_async_copy(
                y_h.at[srcb.at[bb, k]], rows_v.at[b], semg[b]).wait()
            pltpu.async_copy(rows_v.at[b], acc.at[dstb.at[bb, k]],
                             sems[b], add=True)
            if k < LAG:
                @pl.when(j >= LAG)
                def _():
                    scat((blk - 1) % 3, NB - LAG + k, b2).wait()
            else:
                scat(bb, k - LAG, b2).wait()
            if k == NB - LAG - 1:
                @pl.when(blk + 1 < NBLK)
                def _():
                    nxt = (blk + 1) * NB
                    pltpu.make_async_copy(
                        src_h.at[wid, pl.ds(nxt, NB)], srcb.at[nbb], semi).wait()
                    pltpu.make_async_copy(
                        dst_h.at[wid, pl.ds(nxt, NB)], dstb.at[nbb], semi).wait()
            if k + LAG < NB:
                pltpu.async_copy(y_h.at[srcb.at[bb, k + LAG]], rows_v.at[b2],
                                 semg[b2])
            else:
                @pl.when(j + LAG < NCH)
                def _():
                    pltpu.async_copy(y_h.at[srcb.at[nbb, k + LAG - NB]],
                                     rows_v.at[b2], semg[b2])
        return carry

    lax.fori_loop(0, NBLK, block, 0)
    bl = (NBLK - 1) % 3
    for k in range(NB - LAG, NB):
        scat(bl, k, k % NBUF).wait()
    plsc.subcore_barrier()
    pltpu.sync_copy(acc.at[pl.ds(s * ZR, ZR)], out_h.at[c, pl.ds(s * ZR, ZR)])


_seg_kernel = functools.partial(
    pl.kernel,
    out_type=jax.ShapeDtypeStruct((NC, NACC, H), jnp.float32),
    mesh=plsc.VectorSubcoreMesh(core_axis_name="c", subcore_axis_name="s"),
    compiler_params=pltpu.CompilerParams(use_tc_tiling_on_sc=False),
    scratch_types=[
        pltpu.VMEM((3, NB, CH), jnp.int32),
        pltpu.VMEM((3, NB, CH), jnp.int32),
        pltpu.VMEM((NBUF, CH, H), jnp.float32),
        pltpu.VMEM_SHARED((NACC, H), jnp.float32),
    ] + [pltpu.SemaphoreType.DMA] * (2 * NBUF + 1),
)(_seg_body)


N2 = N // 2
NACC2 = NACC // 2


def _bdiag(w_ref):
    w = w_ref[...]
    z = jnp.zeros_like(w)
    return jnp.concatenate(
        [jnp.concatenate([w, z], axis=1), jnp.concatenate([z, w], axis=1)],
        axis=0)


def _dup(v_ref):
    v = v_ref[...]
    return jnp.concatenate([v, v], axis=1)


def _pair_bn(u, g_ref, be_ref):
    mu2 = jnp.mean(u, axis=0, keepdims=True)
    e2 = jnp.mean(u * u, axis=0, keepdims=True)
    mu = 0.5 * (mu2[:, :H] + mu2[:, H:])
    var = 0.5 * (e2[:, :H] + e2[:, H:]) - mu * mu
    muf = jnp.concatenate([mu, mu], axis=1)
    varf = jnp.concatenate([var, var], axis=1)
    return (u - muf) * lax.rsqrt(varf + 1e-5) * _dup(g_ref) + _dup(be_ref)


def _mm_body(x_ref, w_ref, o_ref):
    o_ref[...] = jnp.dot(x_ref[...], w_ref[...],
                         preferred_element_type=jnp.float32)


def _mid_body(y_ref, p_ref, b1_ref, w2_ref, b2_ref, g_ref, be_ref,
              w1n_ref, o_ref):
    h = p_ref[0, :N2, :] + p_ref[1, :N2, :] - y_ref[...] + _dup(b1_ref)
    h = jnp.maximum(h, 0.0)
    u = jnp.dot(h, _bdiag(w2_ref),
                preferred_element_type=jnp.float32) + _dup(b2_ref)
    u = jnp.maximum(u, 0.0)
    hb = _pair_bn(u, g_ref, be_ref)
    o_ref[...] = jnp.dot(hb, _bdiag(w1n_ref), preferred_element_type=jnp.float32)


def _head_body(y_ref, p_ref, b1_ref, w2_ref, b2_ref, g_ref, be_ref,
               l1w_ref, l1b_ref, l2w_ref, l2b_ref, o_ref):
    h = p_ref[0, :N2, :] + p_ref[1, :N2, :] - y_ref[...] + _dup(b1_ref)
    h = jnp.maximum(h, 0.0)
    u = jnp.dot(h, _bdiag(w2_ref),
                preferred_element_type=jnp.float32) + _dup(b2_ref)
    u = jnp.maximum(u, 0.0)
    hb = _pair_bn(u, g_ref, be_ref)
    t = jnp.maximum(
        jnp.dot(hb, _bdiag(l1w_ref), preferred_element_type=jnp.float32)
        + _dup(l1b_ref), 0.0)
    l2w = l2w_ref[...]
    z = jnp.zeros_like(l2w)
    l2d = jnp.concatenate(
        [jnp.concatenate([l2w, z], axis=1), jnp.concatenate([z, l2w], axis=1)],
        axis=0)
    logits = jnp.dot(t, l2d, preferred_element_type=jnp.float32) + _dup(l2b_ref)
    le, lo = logits[:, :C], logits[:, C:]
    me = jnp.max(le, axis=-1, keepdims=True)
    mo = jnp.max(lo, axis=-1, keepdims=True)
    lse_e = me + jnp.log(jnp.sum(jnp.exp(le - me), axis=-1, keepdims=True))
    lse_o = mo + jnp.log(jnp.sum(jnp.exp(lo - mo), axis=-1, keepdims=True))
    o_ref[...] = logits - jnp.concatenate(
        [jnp.broadcast_to(lse_e, (N2, C)), jnp.broadcast_to(lse_o, (N2, C))],
        axis=1)


def _tc(body, out_shape, *args):
    return pl.pallas_call(body, out_shape=out_shape)(*args)


def _padw(w):
    return jnp.pad(w, ((0, 0), (0, HP - H)))


def kernel(x, edge_index, c0_W1, c0_b1, c0_W2, c0_b2, c0_g, c0_be,
           c1_W1, c1_b1, c1_W2, c1_b2, c1_g, c1_be,
           c2_W1, c2_b1, c2_W2, c2_b2, c2_g, c2_be,
           l1_W, l1_b, l2_W, l2_b):
    src, dst = edge_index[0], edge_index[1]
    pad = EPAD - E
    ar = jnp.arange(pad, dtype=jnp.int32)
    src_r = jnp.concatenate([src, ar % N]).reshape(NW, NCH, CH)
    dst_r = jnp.concatenate([dst, N + (ar % JUNK)]).reshape(NW, NCH, CH)

    f32 = jnp.float32
    b1s = [c0_b1.reshape(1, H), c1_b1.reshape(1, H), c2_b1.reshape(1, H)]
    b2s = [c0_b2.reshape(1, H), c1_b2.reshape(1, H), c2_b2.reshape(1, H)]
    gs = [c0_g.reshape(1, H), c1_g.reshape(1, H), c2_g.reshape(1, H)]
    bes = [c0_be.reshape(1, H), c1_be.reshape(1, H), c2_be.reshape(1, H)]
    W2s = [c0_W2, c1_W2, c2_W2]

    y0 = _tc(_mm_body, jax.ShapeDtypeStruct((N, H), f32), x, c0_W1)
    y2 = y0.reshape(N2, 2 * H)

    for i in range(3):
        p = _seg_kernel(y2.reshape(N, H), src_r, dst_r)
        p2 = p.reshape(NC, NACC2, 2 * H)
        if i < 2:
            w1n = c1_W1 if i == 0 else c2_W1
            y2 = _tc(_mid_body, jax.ShapeDtypeStruct((N2, 2 * H), f32),
                     y2, p2, b1s[i], W2s[i], b2s[i], gs[i], bes[i], w1n)
        else:
            out2 = _tc(_head_body, jax.ShapeDtypeStruct((N2, 2 * C), f32),
                       y2, p2, b1s[i], W2s[i], b2s[i], gs[i], bes[i],
                       l1_W, l1_b.reshape(1, H), l2_W, l2_b.reshape(1, C))
    return out2.reshape(N, C)
